# Initial kernel scaffold; baseline (speedup 1.0000x reference)
import jax
import jax.numpy as jnp
from jax.experimental import pallas as pl


def kernel(x, weight, bias, so_cw, so_cb, so_bg, so_bb, so_bm, so_bv, so_ew, so_eb, si_cw, si_cb, si_bg, si_bb, si_bm, si_bv, si_ew, si_eb):
    raise NotImplementedError("write your pallas kernel here")


# speedup vs baseline: 1.1235x; 1.1235x over previous
import functools

import jax
import jax.numpy as jnp
from jax import lax
from jax.experimental import pallas as pl
from jax.experimental.pallas import tpu as pltpu


_INV_SQRT2 = 0.7071067811865476


def _erf(z):
    p = jnp.float32(0.3275911)
    a1 = jnp.float32(0.254829592)
    a2 = jnp.float32(-0.284496736)
    a3 = jnp.float32(1.421413741)
    a4 = jnp.float32(-1.453152027)
    a5 = jnp.float32(1.061405429)
    az = jnp.abs(z)
    t = 1.0 / (1.0 + p * az)
    poly = ((((a5 * t + a4) * t + a3) * t + a2) * t + a1) * t
    e = 1.0 - poly * jnp.exp(-z * z)
    return jnp.where(z < 0, -e, e)


def _gelu_exact(z):
    return 0.5 * z * (1.0 + _erf(z * jnp.float32(_INV_SQRT2)))


def _router_body(xt_ref, cwt_ref, a_ref, b_ref, ewt_ref, eb_ref, vals_ref,
                 idxs_ref):
    xt = xt_ref[...]
    comp = (xt[:, 0, :] * cwt_ref[0, :]
            + xt[:, 1, :] * cwt_ref[1, :]
            + xt[:, 2, :] * cwt_ref[2, :]
            + xt[:, 3, :] * cwt_ref[3, :])
    bn = comp * a_ref[0, :] + b_ref[0, :]
    act = _gelu_exact(bn)
    l = jnp.dot(act, ewt_ref[...], preferred_element_type=jnp.float32)
    l = l + eb_ref[0, :]
    c = l.shape[-1]
    iota = lax.broadcasted_iota(jnp.int32, l.shape, 1)
    m1 = jnp.max(l, axis=-1, keepdims=True)
    i1 = jnp.min(jnp.where(l == m1, iota, c), axis=-1, keepdims=True)
    z = jnp.sum(jnp.exp(l - m1), axis=-1, keepdims=True)
    l2 = jnp.where(iota == i1, -jnp.inf, l)
    m2 = jnp.max(l2, axis=-1, keepdims=True)
    i2 = jnp.min(jnp.where(l2 == m2, iota, c), axis=-1, keepdims=True)
    vals_ref[:, 0:1] = 1.0 / z
    vals_ref[:, 1:2] = jnp.exp(m2 - m1) / z
    idxs_ref[:, 0:1] = i1
    idxs_ref[:, 1:2] = i2


def _run_router(xt, cwt, a, b, ewt, eb, tn):
    n = xt.shape[0]
    g = xt.shape[2]
    c = ewt.shape[1]
    grid = (n // tn,)
    return pl.pallas_call(
        _router_body,
        grid=grid,
        in_specs=[
            pl.BlockSpec((tn, 4, g), lambda t: (t, 0, 0)),
            pl.BlockSpec((4, g), lambda t: (0, 0)),
            pl.BlockSpec((1, g), lambda t: (0, 0)),
            pl.BlockSpec((1, g), lambda t: (0, 0)),
            pl.BlockSpec((g, c), lambda t: (0, 0)),
            pl.BlockSpec((1, c), lambda t: (0, 0)),
        ],
        out_specs=[
            pl.BlockSpec((tn, 2), lambda t: (t, 0)),
            pl.BlockSpec((tn, 2), lambda t: (t, 0)),
        ],
        out_shape=[
            jax.ShapeDtypeStruct((n, 2), jnp.float32),
            jax.ShapeDtypeStruct((n, 2), jnp.int32),
        ],
    )(xt, cwt, a, b, ewt, eb.reshape(1, c))


def _sparse_dense_body(nk, x_ref, w_ref, vo_ref, io_ref, vi_ref, ii_ref,
                       bias_ref, out_ref, pacc, bacc):
    kb = pl.program_id(1)
    kw = x_ref.shape[1]

    @pl.when(kb == 0)
    def _():
        pacc[...] = jnp.zeros_like(pacc)
        bacc[...] = jnp.zeros_like(bacc)

    x_b = x_ref[...]
    ci = kb * kw + lax.broadcasted_iota(jnp.int32, x_b.shape, 1)
    sin = (jnp.where(ci == ii_ref[:, 0:1], vi_ref[:, 0:1], 0.0)
           + jnp.where(ci == ii_ref[:, 1:2], vi_ref[:, 1:2], 0.0))
    w_b = w_ref[...]
    pacc[...] += jnp.dot(x_b, w_b, preferred_element_type=jnp.float32)
    bacc[...] += jnp.dot(x_b * sin, w_b, preferred_element_type=jnp.float32)

    @pl.when(kb == nk - 1)
    def _():
        p = pacc[...]
        co = lax.broadcasted_iota(jnp.int32, p.shape, 1)
        sout = (jnp.where(co == io_ref[:, 0:1], vo_ref[:, 0:1], 0.0)
                + jnp.where(co == io_ref[:, 1:2], vo_ref[:, 1:2], 0.0))
        out_ref[...] = p * sout + bacc[...] + bias_ref[...]


def _run_sparse_dense(xf, w, vo, io, vi, ii, bias2d, tb, kbw):
    n, c1 = xf.shape
    c2 = w.shape[1]
    nk = c1 // kbw
    grid = (n // tb, nk)
    return pl.pallas_call(
        functools.partial(_sparse_dense_body, nk),
        grid=grid,
        in_specs=[
            pl.BlockSpec((tb, kbw), lambda t, k: (t, k)),
            pl.BlockSpec((kbw, c2), lambda t, k: (k, 0)),
            pl.BlockSpec((tb, 2), lambda t, k: (t, 0)),
            pl.BlockSpec((tb, 2), lambda t, k: (t, 0)),
            pl.BlockSpec((tb, 2), lambda t, k: (t, 0)),
            pl.BlockSpec((tb, 2), lambda t, k: (t, 0)),
            pl.BlockSpec((1, c2), lambda t, k: (0, 0)),
        ],
        out_specs=pl.BlockSpec((tb, c2), lambda t, k: (t, 0)),
        out_shape=jax.ShapeDtypeStruct((n, c2), jnp.float32),
        scratch_shapes=[
            pltpu.VMEM((tb, c2), jnp.float32),
            pltpu.VMEM((tb, c2), jnp.float32),
        ],
        compiler_params=pltpu.CompilerParams(
            dimension_semantics=("arbitrary", "arbitrary")),
    )(xf, w, vo, io, vi, ii, bias2d)


def kernel(x, weight, bias, so_cw, so_cb, so_bg, so_bb, so_bm, so_bv, so_ew,
           so_eb, si_cw, si_cb, si_bg, si_bb, si_bm, si_bv, si_ew, si_eb):
    b, l, c1 = x.shape
    c2 = weight.shape[1]
    n = b * l
    g = so_cw.shape[0]

    xf = x.reshape(n, c1)
    xt = xf.reshape(n, g, 4).transpose(0, 2, 1)

    def fold(cw, cb, bg, bb, bm, bv, eb_unused):
        a = bg / jnp.sqrt(bv + 1e-5)
        off = bb - bm * a
        return cw.T, a.reshape(1, g), (off + cb * a).reshape(1, g)

    so_cwt, so_a, so_b = fold(so_cw, so_cb, so_bg, so_bb, so_bm, so_bv, so_eb)
    si_cwt, si_a, si_b = fold(si_cw, si_cb, si_bg, si_bb, si_bm, si_bv, si_eb)
    so_ewt = so_ew.T
    si_ewt = si_ew.T

    def gb(cw, cb, bg, bb, bm, bv, ew, eb):
        xr = xf.reshape(n, g, 4)
        comp = jnp.einsum('ngi,gi->ng', xr, cw) + cb
        bnv = (comp - bm) / jnp.sqrt(bv + 1e-5) * bg + bb
        act = jax.nn.gelu(bnv, approximate=False)
        return act @ ew.T + eb

    sc_o = jax.nn.softmax(
        gb(so_cw, so_cb, so_bg, so_bb, so_bm, so_bv, so_ew,
           so_eb).reshape(b, l, c2), axis=-1)
    vo, io = jax.lax.top_k(sc_o, 2)
    vo = vo.reshape(n, 2)
    io = io.reshape(n, 2).astype(jnp.int32)
    sc_i = jax.nn.softmax(
        gb(si_cw, si_cb, si_bg, si_bb, si_bm, si_bv, si_ew,
           si_eb).reshape(b, l, c1), axis=-1)
    vi, ii = jax.lax.top_k(sc_i, 2)
    vi = vi.reshape(n, 2)
    ii = ii.reshape(n, 2).astype(jnp.int32)

    tb = min(512, n)
    kbw = min(512, c1)
    out = _run_sparse_dense(xf, weight, vo, io, vi, ii,
                            bias.reshape(1, c2), tb, kbw)
    return out.reshape(b, l, c2)


# speedup vs baseline: 10.0176x; 8.9165x over previous
import functools

import jax
import jax.numpy as jnp
from jax import lax
from jax.experimental import pallas as pl
from jax.experimental.pallas import tpu as pltpu


def _top2_body(s_ref, vals_ref, idxs_ref):
    s = s_ref[...]
    c = s.shape[-1]
    iota = lax.broadcasted_iota(jnp.int32, s.shape, 1)
    m1 = jnp.max(s, axis=-1, keepdims=True)
    i1 = jnp.min(jnp.where(s == m1, iota, c), axis=-1, keepdims=True)
    s2 = jnp.where(iota == i1, -jnp.inf, s)
    m2 = jnp.max(s2, axis=-1, keepdims=True)
    i2 = jnp.min(jnp.where(s2 == m2, iota, c), axis=-1, keepdims=True)
    vals_ref[:, 0:1] = m1
    vals_ref[:, 1:2] = m2
    idxs_ref[:, 0:1] = i1
    idxs_ref[:, 1:2] = i2


def _run_top2(scores, tn):
    n, c = scores.shape
    return pl.pallas_call(
        _top2_body,
        grid=(n // tn,),
        in_specs=[pl.BlockSpec((tn, c), lambda t: (t, 0))],
        out_specs=[
            pl.BlockSpec((tn, 2), lambda t: (t, 0)),
            pl.BlockSpec((tn, 2), lambda t: (t, 0)),
        ],
        out_shape=[
            jax.ShapeDtypeStruct((n, 2), jnp.float32),
            jax.ShapeDtypeStruct((n, 2), jnp.int32),
        ],
    )(scores)


def _sparse_dense_body(nk, x_ref, w_ref, vo_ref, io_ref, vi_ref, ii_ref,
                       bias_ref, out_ref, pacc, bacc):
    kb = pl.program_id(1)
    kw = x_ref.shape[1]

    @pl.when(kb == 0)
    def _():
        pacc[...] = jnp.zeros_like(pacc)
        bacc[...] = jnp.zeros_like(bacc)

    x_b = x_ref[...]
    ci = kb * kw + lax.broadcasted_iota(jnp.int32, x_b.shape, 1)
    sin = (jnp.where(ci == ii_ref[:, 0:1], vi_ref[:, 0:1], 0.0)
           + jnp.where(ci == ii_ref[:, 1:2], vi_ref[:, 1:2], 0.0))
    w_b = w_ref[...]
    pacc[...] += jnp.dot(x_b, w_b, preferred_element_type=jnp.float32)
    bacc[...] += jnp.dot(x_b * sin, w_b, preferred_element_type=jnp.float32)

    @pl.when(kb == nk - 1)
    def _():
        p = pacc[...]
        co = lax.broadcasted_iota(jnp.int32, p.shape, 1)
        sout = (jnp.where(co == io_ref[:, 0:1], vo_ref[:, 0:1], 0.0)
                + jnp.where(co == io_ref[:, 1:2], vo_ref[:, 1:2], 0.0))
        out_ref[...] = p * sout + bacc[...] + bias_ref[...]


def _run_sparse_dense(xf, w, vo, io, vi, ii, bias2d, tb, kbw):
    n, c1 = xf.shape
    c2 = w.shape[1]
    nk = c1 // kbw
    grid = (n // tb, nk)
    return pl.pallas_call(
        functools.partial(_sparse_dense_body, nk),
        grid=grid,
        in_specs=[
            pl.BlockSpec((tb, kbw), lambda t, k: (t, k)),
            pl.BlockSpec((kbw, c2), lambda t, k: (k, 0)),
            pl.BlockSpec((tb, 2), lambda t, k: (t, 0)),
            pl.BlockSpec((tb, 2), lambda t, k: (t, 0)),
            pl.BlockSpec((tb, 2), lambda t, k: (t, 0)),
            pl.BlockSpec((tb, 2), lambda t, k: (t, 0)),
            pl.BlockSpec((1, c2), lambda t, k: (0, 0)),
        ],
        out_specs=pl.BlockSpec((tb, c2), lambda t, k: (t, 0)),
        out_shape=jax.ShapeDtypeStruct((n, c2), jnp.float32),
        scratch_shapes=[
            pltpu.VMEM((tb, c2), jnp.float32),
            pltpu.VMEM((tb, c2), jnp.float32),
        ],
        compiler_params=pltpu.CompilerParams(
            dimension_semantics=("arbitrary", "arbitrary")),
    )(xf, w, vo, io, vi, ii, bias2d)


def kernel(x, weight, bias, so_cw, so_cb, so_bg, so_bb, so_bm, so_bv, so_ew,
           so_eb, si_cw, si_cb, si_bg, si_bb, si_bm, si_bv, si_ew, si_eb):
    b, l, c1 = x.shape
    c2 = weight.shape[1]
    n = b * l
    g = so_cw.shape[0]
    xf = x.reshape(n, c1)

    def gb(cw, cb, bg, bb, bm, bv, ew, eb):
        xr = xf.reshape(n, g, 4)
        comp = jnp.einsum('ngi,gi->ng', xr, cw) + cb
        bnv = (comp - bm) / jnp.sqrt(bv + 1e-5) * bg + bb
        act = jax.nn.gelu(bnv, approximate=False)
        return act @ ew.T + eb

    sc_o = jax.nn.softmax(
        gb(so_cw, so_cb, so_bg, so_bb, so_bm, so_bv, so_ew,
           so_eb).reshape(b, l, c2), axis=-1)
    sc_i = jax.nn.softmax(
        gb(si_cw, si_cb, si_bg, si_bb, si_bm, si_bv, si_ew,
           si_eb).reshape(b, l, c1), axis=-1)

    tn = min(256, n)
    vo, io = _run_top2(sc_o.reshape(n, c2), tn)
    vi, ii = _run_top2(sc_i.reshape(n, c1), tn)

    tb = min(512, n)
    kbw = min(512, c1)
    out = _run_sparse_dense(xf, weight, vo, io, vi, ii,
                            bias.reshape(1, c2), tb, kbw)
    return out.reshape(b, l, c2)
